# baseline (speedup 1.0000x reference)
import functools

import jax
import jax.numpy as jnp
from jax import lax
from jax.experimental import pallas as pl
from jax.experimental.pallas import tpu as pltpu
from jax.experimental.pallas import tpu_sc as plsc

L = 16
NC = 2
NS = 16
NW = NC * NS


def kernel(word_emb, pos_emb, type_emb, gamma, beta, input_ids, token_type_ids):
    B, T = input_ids.shape
    V, H = word_emb.shape
    NJ = H // L
    TW = T // NW
    C = 32
    NT = TW // C
    G = NT * B
    inv_h = 1.0 / H
    eps = 1e-12

    ids_flat = input_ids.reshape(B * T)
    tt_flat = token_type_ids.reshape(B * T)
    pos_t = pos_emb[:T]

    mesh = plsc.VectorSubcoreMesh(core_axis_name="c", subcore_axis_name="s")

    @functools.partial(
        pl.kernel,
        mesh=mesh,
        out_type=jax.ShapeDtypeStruct((B * T, H), jnp.float32),
        scratch_types=[
            pltpu.VMEM((C,), jnp.int32),
            pltpu.VMEM((C + L,), jnp.int32),
            pltpu.VMEM((C, H), jnp.float32),
            pltpu.VMEM((C, H), jnp.float32),
            pltpu.VMEM((2, H), jnp.float32),
            pltpu.VMEM((H,), jnp.float32),
            pltpu.VMEM((H,), jnp.float32),
            pltpu.VMEM((H,), jnp.float32),
            pltpu.SemaphoreType.DMA,
        ],
    )
    def sc_embed(word_hbm, pos_hbm, type_hbm, gamma_hbm, beta_hbm,
                 ids_hbm, tt_hbm, out_hbm,
                 idxb, ttb, wb, posb, tbuf, dtv, gv, bv, gsem):
        wid = lax.axis_index("s") * NC + lax.axis_index("c")
        wbase = wid * TW

        pltpu.sync_copy(type_hbm, tbuf)
        pltpu.sync_copy(gamma_hbm, gv)
        pltpu.sync_copy(beta_hbm, bv)
        for j in range(NJ):
            sl = pl.ds(j * L, L)
            dtv[sl] = tbuf[1, sl] - tbuf[0, sl]

        def chunk_body(g, carry):
            tc = g // B
            b = g - tc * B
            t0 = wbase + tc * C
            off = b * T + t0

            @pl.when(b == 0)
            def _load_pos():
                pltpu.sync_copy(pos_hbm.at[pl.ds(t0, C)], posb)

                def fold(i, c2):
                    for j in range(NJ):
                        sl = pl.ds(j * L, L)
                        posb[i, sl] = posb[i, sl] + tbuf[0, sl]
                    return c2
                lax.fori_loop(0, C, fold, 0)

            pltpu.sync_copy(ids_hbm.at[pl.ds(off, C)], idxb)
            pltpu.sync_copy(tt_hbm.at[pl.ds(off, C)], ttb.at[pl.ds(0, C)])
            pltpu.async_copy(word_hbm.at[idxb], wb, gsem).wait()

            def tok(i, c2):
                f = ttb[pl.ds(i, L)][0].astype(jnp.float32)
                s = jnp.zeros((L,), jnp.float32)
                s2 = jnp.zeros((L,), jnp.float32)
                for j in range(NJ):
                    sl = pl.ds(j * L, L)
                    e = wb[i, sl] + posb[i, sl] + f * dtv[sl]
                    wb[i, sl] = e
                    s = s + e
                    s2 = s2 + e * e
                def lane_sum(v):
                    parts = [v[k] for k in range(L)]
                    while len(parts) > 1:
                        parts = [parts[k] + parts[k + 1]
                                 for k in range(0, len(parts), 2)]
                    return parts[0]

                mean = lane_sum(s) * inv_h
                var = lane_sum(s2) * inv_h - mean * mean
                x = var + eps
                xi = lax.bitcast_convert_type(x, jnp.int32)
                yi = jnp.int32(0x5F3759DF) - lax.shift_right_logical(xi, 1)
                y = lax.bitcast_convert_type(yi, jnp.float32)
                for _ in range(3):
                    y = y * (1.5 - 0.5 * x * y * y)
                ma = mean * y
                for j in range(NJ):
                    sl = pl.ds(j * L, L)
                    o = wb[i, sl] * y - ma
                    wb[i, sl] = o * gv[sl] + bv[sl]
                return c2
            lax.fori_loop(0, C, tok, 0)

            pltpu.sync_copy(wb, out_hbm.at[pl.ds(off, C)])
            return carry

        lax.fori_loop(0, G, chunk_body, 0)

    out = sc_embed(word_emb, pos_t, type_emb, gamma, beta, ids_flat, tt_flat)
    return out.reshape(B, T, H)
